# baseline (speedup 1.0000x reference)
import functools

import numpy as np
import jax
import jax.numpy as jnp
from jax import lax
from jax.experimental import pallas as pl
from jax.experimental.pallas import tpu as pltpu
from jax.experimental.pallas import tpu_sc as plsc

_B = 4096
_D = 13
_M = 64
_NT = 26
_V = 100000
_NI = 32
_NW = 32
_BPW = _B // _NW
_BT = 256
_GRID = _B // _BT


def _build_src():
    src = np.full((27, _NI), 351, np.int32)
    p = 0
    for i in range(27):
        for j in range(i):
            src[i, j] = p
            p += 1
    return src


_SRC = _build_src()


def _sc_body(idx_hbm, emb_hbm, out_hbm, idx_v, rows_v, sem):
    wid = lax.axis_index("s") * 2 + lax.axis_index("c")
    b0 = wid * _BPW

    def tbl(t, carry):
        pltpu.sync_copy(idx_hbm.at[t, pl.ds(b0, _BPW)], idx_v)
        off = t * _V
        for j in range(_BPW // 16):
            sl = pl.ds(j * 16, 16)
            idx_v[sl] = idx_v[sl] + off
        pltpu.async_copy(emb_hbm.at[idx_v], rows_v, sem).wait()
        pltpu.sync_copy(rows_v, out_hbm.at[pl.ds(b0, _BPW), t + 1])
        return carry

    lax.fori_loop(0, _NT, tbl, 0)


def _sc_gather(idx, emb2):
    mesh = plsc.VectorSubcoreMesh(core_axis_name="c", subcore_axis_name="s")
    kern = functools.partial(
        pl.kernel,
        mesh=mesh,
        out_type=jax.ShapeDtypeStruct((_B, _NI, _M), jnp.float32),
        scratch_types=[
            pltpu.VMEM((_BPW,), jnp.int32),
            pltpu.VMEM((_BPW, _M), jnp.float32),
            pltpu.SemaphoreType.DMA,
        ],
        compiler_params=pltpu.CompilerParams(use_tc_tiling_on_sc=False),
    )(_sc_body)
    return kern(idx, emb2)


def _tc_body(dx_ref, ly_ref, w0t_ref, b0_ref, w1t_ref, b1_ref, w2t_ref, b2_ref,
             wxt_ref, sf_ref, b3_ref, w4t_ref, b4_ref, w5_ref, b5_ref, out_ref):
    x = dx_ref[...]
    x = jnp.maximum(jnp.dot(x, w0t_ref[...], preferred_element_type=jnp.float32)
                    + b0_ref[...], 0.0)
    x = jnp.maximum(jnp.dot(x, w1t_ref[...], preferred_element_type=jnp.float32)
                    + b1_ref[...], 0.0)
    x3 = jnp.maximum(jnp.dot(x, w2t_ref[...], preferred_element_type=jnp.float32)
                     + b2_ref[...], 0.0)

    ly = ly_ref[...]
    ii = lax.broadcasted_iota(jnp.int32, (_BT, _NI, _M), 1)
    t3 = jnp.where(ii == 0, x3[:, None, :], jnp.where(ii < 27, ly, 0.0))
    t3b = t3.astype(jnp.bfloat16)
    z = lax.dot_general(t3b, t3b, (((2,), (2,)), ((0,), (0,))),
                        preferred_element_type=jnp.float32)

    y = jnp.dot(x3, wxt_ref[...], preferred_element_type=jnp.float32) + b3_ref[...]
    sf = sf_ref[...]
    for i in range(1, 27):
        y = y + jnp.dot(z[:, i, :].astype(jnp.bfloat16), sf[i],
                        preferred_element_type=jnp.float32)

    t1 = jnp.maximum(y, 0.0)
    t2 = jnp.maximum(jnp.dot(t1, w4t_ref[...], preferred_element_type=jnp.float32)
                     + b4_ref[...], 0.0)
    logit = jnp.sum(t2 * w5_ref[...], axis=1, keepdims=True) + b5_ref[...]
    out_ref[...] = 1.0 / (1.0 + jnp.exp(-logit))


def _w_spec(shape):
    nd = len(shape)
    return pl.BlockSpec(shape, lambda i, _n=nd: (0,) * _n)


_TC_GRID = (_GRID,)
_TC_OUT_SHAPE = jax.ShapeDtypeStruct((_B, 1), jnp.float32)
_TC_IN_SPECS = [
    pl.BlockSpec((_BT, _D), lambda i: (i, 0)),
    pl.BlockSpec((_BT, _NI, _M), lambda i: (i, 0, 0)),
    _w_spec((_D, 512)),
    _w_spec((1, 512)),
    _w_spec((512, 256)),
    _w_spec((1, 256)),
    _w_spec((256, _M)),
    _w_spec((1, _M)),
    _w_spec((_M, 512)),
    _w_spec((27, _NI, 512)),
    _w_spec((1, 512)),
    _w_spec((512, 256)),
    _w_spec((1, 256)),
    _w_spec((1, 256)),
    _w_spec((1, 1)),
]
_TC_OUT_SPEC = pl.BlockSpec((_BT, 1), lambda i: (i, 0))


def _tc_call(interpret_args, *ops):
    return pl.pallas_call(
        _tc_body,
        grid=_TC_GRID,
        in_specs=_TC_IN_SPECS,
        out_specs=_TC_OUT_SPEC,
        out_shape=_TC_OUT_SHAPE,
        **interpret_args,
    )(*ops)


def kernel(dense_x, lS_o, lS_i, emb,
           bot_W0, bot_b0, bot_W1, bot_b1, bot_W2, bot_b2,
           top_W0, top_b0, top_W1, top_b1, top_W2, top_b2):
    idx = lS_i.astype(jnp.int32)
    emb2 = emb.reshape(_NT * _V, _M)
    ly3 = _sc_gather(idx, emb2)

    w0t = bot_W0.T
    w1t = bot_W1.T
    w2t = bot_W2.T
    wxt = top_W0[:, :_M].T
    wzt = top_W0[:, _M:].T
    wzt_pad = jnp.concatenate([wzt, jnp.zeros((1, 512), jnp.float32)], axis=0)
    sfold = jnp.take(wzt_pad, _SRC.reshape(-1), axis=0)
    sfold = sfold.reshape(27, _NI, 512).astype(jnp.bfloat16)
    w4t = top_W1.T

    ops = (dense_x, ly3,
           w0t, bot_b0[None, :], w1t, bot_b1[None, :], w2t, bot_b2[None, :],
           wxt, sfold, top_b0[None, :], w4t, top_b1[None, :],
           top_W2, top_b2[None, :])
    return _tc_call({}, *ops)


# speedup vs baseline: 1.6105x; 1.6105x over previous
import functools

import numpy as np
import jax
import jax.numpy as jnp
from jax import lax
from jax.experimental import pallas as pl
from jax.experimental.pallas import tpu as pltpu
from jax.experimental.pallas import tpu_sc as plsc

_B = 4096
_D = 13
_M = 64
_NT = 26
_V = 100000
_NI = 32
_NW = 32
_BPW = _B // _NW
_BT = 256
_GRID = _B // _BT


def _build_src():
    src = np.full((27, _NI), 351, np.int32)
    p = 0
    for i in range(27):
        for j in range(i):
            src[i, j] = p
            p += 1
    return src


_SRC = _build_src()


def _sc_body(idx_hbm, emb_hbm, out_hbm, idx_v, idx_sh, idx_s, rows_v, sem):
    sid = lax.axis_index("s")
    wid = sid * 2 + lax.axis_index("c")
    b0 = wid * _BPW

    def stage_idx(t):
        pltpu.sync_copy(idx_hbm.at[t, pl.ds(b0, _BPW)], idx_v)
        pltpu.sync_copy(idx_v, idx_sh.at[sid])
        pltpu.sync_copy(idx_sh.at[sid], idx_s.at[t % 2])

    stage_idx(0)

    def tbl(t, carry):
        buf = t % 2

        def row(k, carry2):
            r = idx_s[buf, k]
            pltpu.async_copy(emb_hbm.at[t, r], rows_v.at[buf, k], sem)
            return carry2

        lax.fori_loop(0, _BPW, row, 0, unroll=4)

        @pl.when(t + 1 < _NT)
        def _():
            stage_idx(t + 1)

        pltpu.make_async_copy(
            emb_hbm.at[0, pl.ds(0, _BPW)], rows_v.at[buf], sem).wait()
        pltpu.sync_copy(rows_v.at[buf], out_hbm.at[pl.ds(b0, _BPW), t + 1])
        return carry

    lax.fori_loop(0, _NT, tbl, 0)


def _sc_gather(idx, emb3):
    mesh = plsc.VectorSubcoreMesh(core_axis_name="c", subcore_axis_name="s")
    kern = functools.partial(
        pl.kernel,
        mesh=mesh,
        out_type=jax.ShapeDtypeStruct((_B, _NI, _M), jnp.float32),
        scratch_types=[
            pltpu.VMEM((_BPW,), jnp.int32),
            pltpu.MemorySpace.VMEM_SHARED((16, _BPW), jnp.int32),
            pltpu.SMEM((2, _BPW), jnp.int32),
            pltpu.VMEM((2, _BPW, _M), jnp.float32),
            pltpu.SemaphoreType.DMA,
        ],
    )(_sc_body)
    return kern(idx, emb3)


def _tc_body(dx_ref, ly_ref, w0t_ref, b0_ref, w1t_ref, b1_ref, w2t_ref, b2_ref,
             wxt_ref, sf_ref, b3_ref, w4t_ref, b4_ref, w5_ref, b5_ref, out_ref):
    x = dx_ref[...]
    x = jnp.maximum(jnp.dot(x, w0t_ref[...], preferred_element_type=jnp.float32)
                    + b0_ref[...], 0.0)
    x = jnp.maximum(jnp.dot(x, w1t_ref[...], preferred_element_type=jnp.float32)
                    + b1_ref[...], 0.0)
    x3 = jnp.maximum(jnp.dot(x, w2t_ref[...], preferred_element_type=jnp.float32)
                     + b2_ref[...], 0.0)

    ly = ly_ref[...]
    ii = lax.broadcasted_iota(jnp.int32, (_BT, _NI, _M), 1)
    t3 = jnp.where(ii == 0, x3[:, None, :], jnp.where(ii < 27, ly, 0.0))
    t3b = t3.astype(jnp.bfloat16)
    z = lax.dot_general(t3b, t3b, (((2,), (2,)), ((0,), (0,))),
                        preferred_element_type=jnp.float32)

    y = jnp.dot(x3, wxt_ref[...], preferred_element_type=jnp.float32) + b3_ref[...]
    sf = sf_ref[...]
    for i in range(1, 27):
        y = y + jnp.dot(z[:, i, :].astype(jnp.bfloat16), sf[i],
                        preferred_element_type=jnp.float32)

    t1 = jnp.maximum(y, 0.0)
    t2 = jnp.maximum(jnp.dot(t1, w4t_ref[...], preferred_element_type=jnp.float32)
                     + b4_ref[...], 0.0)
    logit = jnp.sum(t2 * w5_ref[...], axis=1, keepdims=True) + b5_ref[...]
    out_ref[...] = 1.0 / (1.0 + jnp.exp(-logit))


def _w_spec(shape):
    nd = len(shape)
    return pl.BlockSpec(shape, lambda i, _n=nd: (0,) * _n)


_TC_GRID = (_GRID,)
_TC_OUT_SHAPE = jax.ShapeDtypeStruct((_B, 1), jnp.float32)
_TC_IN_SPECS = [
    pl.BlockSpec((_BT, _D), lambda i: (i, 0)),
    pl.BlockSpec((_BT, _NI, _M), lambda i: (i, 0, 0)),
    _w_spec((_D, 512)),
    _w_spec((1, 512)),
    _w_spec((512, 256)),
    _w_spec((1, 256)),
    _w_spec((256, _M)),
    _w_spec((1, _M)),
    _w_spec((_M, 512)),
    _w_spec((27, _NI, 512)),
    _w_spec((1, 512)),
    _w_spec((512, 256)),
    _w_spec((1, 256)),
    _w_spec((1, 256)),
    _w_spec((1, 1)),
]
_TC_OUT_SPEC = pl.BlockSpec((_BT, 1), lambda i: (i, 0))


def _tc_call(interpret_args, *ops):
    return pl.pallas_call(
        _tc_body,
        grid=_TC_GRID,
        in_specs=_TC_IN_SPECS,
        out_specs=_TC_OUT_SPEC,
        out_shape=_TC_OUT_SHAPE,
        **interpret_args,
    )(*ops)


def kernel(dense_x, lS_o, lS_i, emb,
           bot_W0, bot_b0, bot_W1, bot_b1, bot_W2, bot_b2,
           top_W0, top_b0, top_W1, top_b1, top_W2, top_b2):
    idx = lS_i.astype(jnp.int32)
    ly3 = _sc_gather(idx, emb)

    w0t = bot_W0.T
    w1t = bot_W1.T
    w2t = bot_W2.T
    wxt = top_W0[:, :_M].T
    wzt = top_W0[:, _M:].T
    wzt_pad = jnp.concatenate([wzt, jnp.zeros((1, 512), jnp.float32)], axis=0)
    sfold = jnp.take(wzt_pad, _SRC.reshape(-1), axis=0)
    sfold = sfold.reshape(27, _NI, 512).astype(jnp.bfloat16)
    w4t = top_W1.T

    ops = (dense_x, ly3,
           w0t, bot_b0[None, :], w1t, bot_b1[None, :], w2t, bot_b2[None, :],
           wxt, sfold, top_b0[None, :], w4t, top_b1[None, :],
           top_W2, top_b2[None, :])
    return _tc_call({}, *ops)
